# Initial kernel scaffold; baseline (speedup 1.0000x reference)
import jax
import jax.numpy as jnp
from jax.experimental import pallas as pl


def kernel(x, W1, b1, W2, b2, W3, b3, codebook):
    raise NotImplementedError("write your pallas kernel here")


# speedup vs baseline: 1.0096x; 1.0096x over previous
import functools

import jax
import jax.numpy as jnp
from jax.experimental import pallas as pl


def _conv(x, w, b, stride=2):
    y = jax.lax.conv_general_dilated(
        x, w, window_strides=(stride, stride), padding=[(1, 1), (1, 1)],
        dimension_numbers=('NCHW', 'OIHW', 'NCHW'))
    return y + b[None, :, None, None]


def _vq_body(fl2_ref, cb_ref, a_ref, b_ref, idx_ref):
    fl2 = fl2_ref[...]
    c = cb_ref[...]
    a = a_ref[...]
    bsq = b_ref[...]
    dots2 = jax.lax.dot_general(fl2, c, (((1,), (1,)), ((), ())),
                                preferred_element_type=jnp.float32)
    dist = (a[:, None] + bsq[None, :]) - dots2
    m = jnp.min(dist, axis=1, keepdims=True)
    k = dist.shape[1]
    iota = jax.lax.broadcasted_iota(jnp.int32, dist.shape, 1)
    idx_ref[...] = jnp.min(jnp.where(dist == m, iota, k), axis=1).astype(jnp.int32)


def _vq_argmin(fl2, codebook, a, bsq, block_m=256):
    m, d = fl2.shape
    k = codebook.shape[0]
    return pl.pallas_call(
        _vq_body,
        grid=(m // block_m,),
        in_specs=[pl.BlockSpec((block_m, d), lambda i: (i, 0)),
                  pl.BlockSpec((k, d), lambda i: (0, 0)),
                  pl.BlockSpec((block_m,), lambda i: (i,)),
                  pl.BlockSpec((k,), lambda i: (0,))],
        out_specs=pl.BlockSpec((block_m,), lambda i: (i,)),
        out_shape=jax.ShapeDtypeStruct((m,), jnp.int32),
    )(fl2, codebook, a, bsq)


def kernel(x, W1, b1, W2, b2, W3, b3, codebook):
    z = jax.nn.relu(_conv(x, W1, b1))
    z = jax.nn.relu(_conv(z, W2, b2))
    z = jax.nn.relu(_conv(z, W3, b3))
    zc = jnp.transpose(z, (0, 2, 3, 1))
    d = codebook.shape[1]
    flat = zc.reshape(-1, d)
    a = jnp.sum(flat ** 2, axis=1)
    bsq = jnp.sum(codebook ** 2, axis=1)
    fl2 = (2.0 * flat).astype(jnp.bfloat16)
    idx = _vq_argmin(fl2, codebook, a, bsq)
    return idx.reshape(x.shape[0], -1), jnp.array(0.0, dtype=jnp.float32)


# speedup vs baseline: 1.0105x; 1.0009x over previous
import jax
import jax.numpy as jnp
from jax.experimental import pallas as pl


def _conv(x, w, b, stride=2):
    y = jax.lax.conv_general_dilated(
        x, w, window_strides=(stride, stride), padding=[(1, 1), (1, 1)],
        dimension_numbers=('NCHW', 'OIHW', 'NCHW'))
    return y + b[None, :, None, None]


def _vq_body(fl2_ref, cb_ref, a_ref, b_ref, idx_ref):
    fl2 = fl2_ref[...]
    c = cb_ref[...]
    a = a_ref[...]
    bsq = b_ref[...]
    dots2 = jax.lax.dot_general(fl2, c, (((1,), (1,)), ((), ())),
                                preferred_element_type=jnp.float32)
    dist = (a[:, None] + bsq[None, :]) - dots2
    m = jnp.min(dist, axis=1, keepdims=True)
    k = dist.shape[1]
    iota = jax.lax.broadcasted_iota(jnp.int32, dist.shape, 1)
    idx_ref[...] = jnp.min(jnp.where(dist == m, iota, k), axis=1).astype(jnp.int32)


def _vq_argmin(fl2, codebook, a, bsq, block_m=256):
    m, d = fl2.shape
    k = codebook.shape[0]
    return pl.pallas_call(
        _vq_body,
        grid=(m // block_m,),
        in_specs=[pl.BlockSpec((block_m, d), lambda i: (i, 0)),
                  pl.BlockSpec((k, d), lambda i: (0, 0)),
                  pl.BlockSpec((block_m,), lambda i: (i,)),
                  pl.BlockSpec((k,), lambda i: (0,))],
        out_specs=pl.BlockSpec((block_m,), lambda i: (i,)),
        out_shape=jax.ShapeDtypeStruct((m,), jnp.int32),
    )(fl2, codebook, a, bsq)


def kernel(x, W1, b1, W2, b2, W3, b3, codebook):
    z = jax.nn.relu(_conv(x, W1, b1))
    z = jax.nn.relu(_conv(z, W2, b2))
    z = jax.nn.relu(_conv(z, W3, b3))
    zc = jnp.transpose(z, (0, 2, 3, 1))
    d = codebook.shape[1]
    flat = zc.reshape(-1, d)
    a = jnp.sum(flat ** 2, axis=1)
    bsq = jnp.sum(codebook ** 2, axis=1)
    fl2 = (2.0 * flat).astype(jnp.bfloat16)
    idx = _vq_argmin(fl2, codebook, a, bsq)
    return idx.reshape(x.shape[0], -1), jnp.array(0.0, dtype=jnp.float32)
